# baseline (speedup 1.0000x reference)
import functools

import jax
import jax.numpy as jnp
from jax import lax
from jax.experimental import pallas as pl
from jax.experimental.pallas import tpu as pltpu
from jax.experimental.pallas import tpu_sc as plsc

B = 8
IMG = 512 * 512
N = B * IMG
NC = 2
NS = 16
NW = NC * NS
PER_TILE = N // NW
TILES_PER_B = NW // B
LANES = 16
CHUNK = 16384
NCHUNK = PER_TILE // CHUNK
VPC = CHUNK // LANES

_mesh = plsc.VectorSubcoreMesh(core_axis_name="c", subcore_axis_name="s")


@functools.partial(
    pl.kernel,
    mesh=_mesh,
    out_type=jax.ShapeDtypeStruct((NW, 8, LANES), jnp.float32),
    compiler_params=pltpu.CompilerParams(needs_layout_passes=False),
    scratch_types=[
        pltpu.VMEM((CHUNK,), jnp.float32),
        pltpu.VMEM((CHUNK,), jnp.float32),
        pltpu.VMEM((CHUNK,), jnp.int32),
        pltpu.VMEM((5 * LANES,), jnp.float32),
        pltpu.VMEM((5 * LANES,), jnp.float32),
        pltpu.VMEM((8, LANES), jnp.float32),
    ],
)
def _partial_sums(out_hbm, tgt_hbm, msk_hbm, part_hbm, obuf, tbuf, mbuf,
                  bkt_s, bkt_c, pvec):
    wid = lax.axis_index("s") * NC + lax.axis_index("c")
    base = wid * PER_TILE
    zero = jnp.zeros((LANES,), jnp.float32)
    one = jnp.ones((LANES,), jnp.float32)
    lanes = lax.iota(jnp.int32, LANES)

    for k in range(5):
        bkt_s[pl.ds(k * LANES, LANES)] = zero
        bkt_c[pl.ds(k * LANES, LANES)] = zero

    def chunk_body(ci, carry):
        off = base + ci * CHUNK
        pltpu.sync_copy(out_hbm.at[pl.ds(off, CHUNK)], obuf)
        pltpu.sync_copy(tgt_hbm.at[pl.ds(off, CHUNK)], tbuf)
        pltpu.sync_copy(msk_hbm.at[pl.ds(off, CHUNK)], mbuf)

        def vec_body(j, carry2):
            sl = pl.ds(j * LANES, LANES)
            o = obuf[sl]
            t = tbuf[sl]
            m = mbuf[sl]
            d = o - t
            d2 = d * d
            idx = lax.shift_left(m, 4) + lanes
            plsc.addupdate_scatter(bkt_s, [idx], d2)
            plsc.addupdate_scatter(bkt_c, [idx], one)
            return carry2

        return lax.fori_loop(0, VPC, vec_body, carry)

    lax.fori_loop(0, NCHUNK, chunk_body, 0)
    for i in range(1, 5):
        pvec[i - 1, :] = bkt_s[pl.ds(i * LANES, LANES)]
        pvec[i + 3, :] = bkt_c[pl.ds(i * LANES, LANES)]
    pltpu.sync_copy(pvec, part_hbm.at[wid])


@functools.partial(
    pl.kernel,
    mesh=_mesh,
    out_type=jax.ShapeDtypeStruct((LANES,), jnp.float32),
    scratch_types=[
        pltpu.VMEM((NW, 8, LANES), jnp.float32),
        pltpu.VMEM((LANES,), jnp.float32),
    ],
)
def _combine(part_hbm, out_hbm, pbuf, obuf):
    wid = lax.axis_index("s") * NC + lax.axis_index("c")

    @pl.when(wid == 0)
    def _():
        pltpu.sync_copy(part_hbm, pbuf)
        zero = jnp.zeros((LANES,), jnp.float32)
        lanes = lax.iota(jnp.int32, LANES)
        perms = [jnp.reshape(jnp.bitwise_xor(lanes, d), (LANES, 1))
                 for d in (1, 2, 4, 8)]
        dn = lax.GatherDimensionNumbers(
            offset_dims=(), collapsed_slice_dims=(0,), start_index_map=(0,))

        def lane_sum(v):
            for p in perms:
                v = v + lax.gather(v, p, dn, slice_sizes=(1,),
                                   mode=lax.GatherScatterMode.PROMISE_IN_BOUNDS)
            return v

        lossv = zero
        for b in range(B):
            t0 = TILES_PER_B * b
            for i in range(4):
                v = (pbuf[t0 + 0, i, :] + pbuf[t0 + 1, i, :]
                     + pbuf[t0 + 2, i, :] + pbuf[t0 + 3, i, :])
                c = (pbuf[t0 + 0, i + 4, :] + pbuf[t0 + 1, i + 4, :]
                     + pbuf[t0 + 2, i + 4, :] + pbuf[t0 + 3, i + 4, :])
                sv = lane_sum(v)
                cv = lane_sum(c)
                contrib = jnp.where(cv > 0.0, sv / jnp.maximum(cv, 1.0), zero)
                lossv = lossv + contrib
        obuf[...] = lossv * jnp.float32(1.0 / B)
        pltpu.sync_copy(obuf, out_hbm)


def kernel(output, target, mask):
    o = output.reshape(N)
    t = target.reshape(N)
    m = mask.reshape(N)
    part = _partial_sums(o, t, m)
    res = _combine(part)
    return res[0]


# speedup vs baseline: 1.2068x; 1.2068x over previous
import functools

import jax
import jax.numpy as jnp
from jax import lax
from jax.experimental import pallas as pl
from jax.experimental.pallas import tpu as pltpu
from jax.experimental.pallas import tpu_sc as plsc

B = 8
IMG = 512 * 512
N = B * IMG
NC = 2
NS = 16
NW = NC * NS
PER_TILE = N // NW
TILES_PER_B = NW // B
LANES = 16
CHUNK = 8192
NCHUNK = PER_TILE // CHUNK
UNROLL = 4
VPC = CHUNK // (LANES * UNROLL)

_mesh = plsc.VectorSubcoreMesh(core_axis_name="c", subcore_axis_name="s")


@functools.partial(
    pl.kernel,
    mesh=_mesh,
    out_type=jax.ShapeDtypeStruct((NW, 8, LANES), jnp.float32),
    compiler_params=pltpu.CompilerParams(needs_layout_passes=False),
    scratch_types=[
        pltpu.VMEM((2, CHUNK), jnp.float32),
        pltpu.VMEM((2, CHUNK), jnp.float32),
        pltpu.VMEM((2, CHUNK), jnp.int32),
        pltpu.VMEM((8, LANES), jnp.float32),
        pltpu.SemaphoreType.DMA,
        pltpu.SemaphoreType.DMA,
    ],
)
def _partial_sums(out_hbm, tgt_hbm, msk_hbm, part_hbm, obuf, tbuf, mbuf,
                  pvec, sem0, sem1):
    wid = lax.axis_index("s") * NC + lax.axis_index("c")
    base = wid * PER_TILE
    zero = jnp.zeros((LANES,), jnp.float32)
    one = jnp.ones((LANES,), jnp.float32)
    sems = (sem0, sem1)

    def start_fetch(ci):
        p = ci % 2
        off = base + ci * CHUNK
        sl = pl.ds(off, CHUNK)
        return [
            pltpu.async_copy(out_hbm.at[sl], obuf.at[p], sems[p]),
            pltpu.async_copy(tgt_hbm.at[sl], tbuf.at[p], sems[p]),
            pltpu.async_copy(msk_hbm.at[sl], mbuf.at[p], sems[p]),
        ]

    inflight = start_fetch(0)
    acc = (zero,) * 8
    for ci in range(NCHUNK):
        p = ci % 2
        for cp in inflight:
            cp.wait()
        if ci + 1 < NCHUNK:
            inflight = start_fetch(ci + 1)

        def vec_body(j, carry, p=p):
            a1, a2, a3, a4, c1, c2, c3, c4 = carry
            for k in range(UNROLL):
                sl = pl.ds(j * (LANES * UNROLL) + k * LANES, LANES)
                o = obuf[p, sl]
                t = tbuf[p, sl]
                m = mbuf[p, sl]
                d = o - t
                d2 = d * d
                f1 = jnp.where(m == 1, one, zero)
                f2 = jnp.where(m == 2, one, zero)
                f3 = jnp.where(m == 3, one, zero)
                f4 = jnp.where(m == 4, one, zero)
                a1 = a1 + d2 * f1
                a2 = a2 + d2 * f2
                a3 = a3 + d2 * f3
                a4 = a4 + d2 * f4
                c1 = c1 + f1
                c2 = c2 + f2
                c3 = c3 + f3
                c4 = c4 + f4
            return (a1, a2, a3, a4, c1, c2, c3, c4)

        acc = lax.fori_loop(0, VPC, vec_body, acc)

    for k in range(8):
        pvec[k, :] = acc[k]
    pltpu.sync_copy(pvec, part_hbm.at[wid])


@functools.partial(
    pl.kernel,
    mesh=_mesh,
    out_type=jax.ShapeDtypeStruct((LANES,), jnp.float32),
    scratch_types=[
        pltpu.VMEM((NW, 8, LANES), jnp.float32),
        pltpu.VMEM((LANES,), jnp.float32),
    ],
)
def _combine(part_hbm, out_hbm, pbuf, obuf):
    wid = lax.axis_index("s") * NC + lax.axis_index("c")

    @pl.when(wid == 0)
    def _():
        pltpu.sync_copy(part_hbm, pbuf)
        zero = jnp.zeros((LANES,), jnp.float32)
        lanes = lax.iota(jnp.int32, LANES)
        perms = [jnp.reshape(jnp.bitwise_xor(lanes, d), (LANES, 1))
                 for d in (1, 2, 4, 8)]
        dn = lax.GatherDimensionNumbers(
            offset_dims=(), collapsed_slice_dims=(0,), start_index_map=(0,))

        def lane_sum(v):
            for p in perms:
                v = v + lax.gather(v, p, dn, slice_sizes=(1,),
                                   mode=lax.GatherScatterMode.PROMISE_IN_BOUNDS)
            return v

        lossv = zero
        for b in range(B):
            t0 = TILES_PER_B * b
            for i in range(4):
                v = (pbuf[t0 + 0, i, :] + pbuf[t0 + 1, i, :]
                     + pbuf[t0 + 2, i, :] + pbuf[t0 + 3, i, :])
                c = (pbuf[t0 + 0, i + 4, :] + pbuf[t0 + 1, i + 4, :]
                     + pbuf[t0 + 2, i + 4, :] + pbuf[t0 + 3, i + 4, :])
                sv = lane_sum(v)
                cv = lane_sum(c)
                contrib = jnp.where(cv > 0.0, sv / jnp.maximum(cv, 1.0), zero)
                lossv = lossv + contrib
        obuf[...] = lossv * jnp.float32(1.0 / B)
        pltpu.sync_copy(obuf, out_hbm)


def kernel(output, target, mask):
    o = output.reshape(N)
    t = target.reshape(N)
    m = mask.reshape(N)
    part = _partial_sums(o, t, m)
    res = _combine(part)
    return res[0]


# speedup vs baseline: 2.2355x; 1.8525x over previous
import functools

import jax
import jax.numpy as jnp
from jax import lax
from jax.experimental import pallas as pl
from jax.experimental.pallas import tpu as pltpu
from jax.experimental.pallas import tpu_sc as plsc

B = 8
IMG = 512 * 512
N = B * IMG
NC = 2
NS = 16
NW = NC * NS
PER_TILE = N // NW
TILES_PER_B = NW // B
LANES = 16
ROWS = 512
COLS = 512
ROWS_PER_TILE = ROWS // TILES_PER_B
CR = 16
CHUNK = CR * COLS
NCHUNK = ROWS_PER_TILE // CR
UNROLL = 4
VPC = CHUNK // (LANES * UNROLL)

_mesh = plsc.VectorSubcoreMesh(core_axis_name="c", subcore_axis_name="s")


@functools.partial(
    pl.kernel,
    mesh=_mesh,
    out_type=jax.ShapeDtypeStruct((NW, 8, LANES), jnp.float32),
    compiler_params=pltpu.CompilerParams(
        needs_layout_passes=False, use_tc_tiling_on_sc=True),
    scratch_types=[
        pltpu.VMEM((2, CR, COLS), jnp.float32),
        pltpu.VMEM((2, CR, COLS), jnp.float32),
        pltpu.VMEM((2, CR, COLS), jnp.int32),
        pltpu.VMEM((8, LANES), jnp.float32),
        pltpu.SemaphoreType.DMA,
        pltpu.SemaphoreType.DMA,
    ],
)
def _partial_sums(out_hbm, tgt_hbm, msk_hbm, part_hbm, obuf, tbuf, mbuf,
                  pvec, sem0, sem1):
    wid = lax.axis_index("s") * NC + lax.axis_index("c")
    b = wid // TILES_PER_B
    r_base = (wid % TILES_PER_B) * ROWS_PER_TILE
    zero = jnp.zeros((LANES,), jnp.float32)
    one = jnp.ones((LANES,), jnp.float32)
    sems = (sem0, sem1)

    def start_fetch(ci):
        p = ci % 2
        sl = pl.ds(r_base + ci * CR, CR)
        return [
            pltpu.async_copy(out_hbm.at[b, sl, :], obuf.at[p], sems[p]),
            pltpu.async_copy(tgt_hbm.at[b, sl, :], tbuf.at[p], sems[p]),
            pltpu.async_copy(msk_hbm.at[b, sl, :], mbuf.at[p], sems[p]),
        ]

    inflight = start_fetch(0)
    acc = (zero,) * 8
    for ci in range(NCHUNK):
        p = ci % 2
        for cp in inflight:
            cp.wait()
        if ci + 1 < NCHUNK:
            inflight = start_fetch(ci + 1)

        def vec_body(j, carry, p=p):
            a1, a2, a3, a4, c1, c2, c3, c4 = carry
            i = j // 8
            c0 = (j % 8) * (LANES * UNROLL)
            for k in range(UNROLL):
                sl = pl.ds(c0 + k * LANES, LANES)
                o = obuf[p, i, sl]
                t = tbuf[p, i, sl]
                m = mbuf[p, i, sl]
                d = o - t
                d2 = d * d
                f1 = jnp.where(m == 1, one, zero)
                f2 = jnp.where(m == 2, one, zero)
                f3 = jnp.where(m == 3, one, zero)
                f4 = jnp.where(m == 4, one, zero)
                a1 = a1 + d2 * f1
                a2 = a2 + d2 * f2
                a3 = a3 + d2 * f3
                a4 = a4 + d2 * f4
                c1 = c1 + f1
                c2 = c2 + f2
                c3 = c3 + f3
                c4 = c4 + f4
            return (a1, a2, a3, a4, c1, c2, c3, c4)

        acc = lax.fori_loop(0, VPC, vec_body, acc)

    for k in range(8):
        pvec[k, :] = acc[k]
    pltpu.sync_copy(pvec, part_hbm.at[wid])


@functools.partial(
    pl.kernel,
    mesh=_mesh,
    out_type=jax.ShapeDtypeStruct((LANES,), jnp.float32),
    scratch_types=[
        pltpu.VMEM((NW, 8, LANES), jnp.float32),
        pltpu.VMEM((LANES,), jnp.float32),
    ],
)
def _combine(part_hbm, out_hbm, pbuf, obuf):
    wid = lax.axis_index("s") * NC + lax.axis_index("c")

    @pl.when(wid == 0)
    def _():
        pltpu.sync_copy(part_hbm, pbuf)
        zero = jnp.zeros((LANES,), jnp.float32)
        lanes = lax.iota(jnp.int32, LANES)
        perms = [jnp.reshape(jnp.bitwise_xor(lanes, d), (LANES, 1))
                 for d in (1, 2, 4, 8)]
        dn = lax.GatherDimensionNumbers(
            offset_dims=(), collapsed_slice_dims=(0,), start_index_map=(0,))

        def lane_sum(v):
            for p in perms:
                v = v + lax.gather(v, p, dn, slice_sizes=(1,),
                                   mode=lax.GatherScatterMode.PROMISE_IN_BOUNDS)
            return v

        lossv = zero
        for b in range(B):
            t0 = TILES_PER_B * b
            for i in range(4):
                v = (pbuf[t0 + 0, i, :] + pbuf[t0 + 1, i, :]
                     + pbuf[t0 + 2, i, :] + pbuf[t0 + 3, i, :])
                c = (pbuf[t0 + 0, i + 4, :] + pbuf[t0 + 1, i + 4, :]
                     + pbuf[t0 + 2, i + 4, :] + pbuf[t0 + 3, i + 4, :])
                sv = lane_sum(v)
                cv = lane_sum(c)
                contrib = jnp.where(cv > 0.0, sv / jnp.maximum(cv, 1.0), zero)
                lossv = lossv + contrib
        obuf[...] = lossv * jnp.float32(1.0 / B)
        pltpu.sync_copy(obuf, out_hbm)


def kernel(output, target, mask):
    part = _partial_sums(output, target, mask)
    res = _combine(part)
    return res[0]


# speedup vs baseline: 2.3213x; 1.0384x over previous
import functools

import jax
import jax.numpy as jnp
from jax import lax
from jax.experimental import pallas as pl
from jax.experimental.pallas import tpu as pltpu
from jax.experimental.pallas import tpu_sc as plsc

B = 8
NC = 2
NS = 16
NW = NC * NS
B_PER_CORE = B // NC
TILES_PER_B = NS // B_PER_CORE
LANES = 16
ROWS = 512
COLS = 512
ROWS_PER_TILE = ROWS // TILES_PER_B
CR = 16
CHUNK = CR * COLS
NCHUNK = ROWS_PER_TILE // CR
UNROLL = 4
VPC = CHUNK // (LANES * UNROLL)

_mesh = plsc.VectorSubcoreMesh(core_axis_name="c", subcore_axis_name="s")


@functools.partial(
    pl.kernel,
    mesh=_mesh,
    out_type=(
        jax.ShapeDtypeStruct((NW, 8, LANES), jnp.float32),
        jax.ShapeDtypeStruct((NC, LANES), jnp.float32),
    ),
    compiler_params=pltpu.CompilerParams(
        needs_layout_passes=False, use_tc_tiling_on_sc=True),
    scratch_types=[
        pltpu.VMEM((2, CR, COLS), jnp.float32),
        pltpu.VMEM((2, CR, COLS), jnp.float32),
        pltpu.VMEM((2, CR, COLS), jnp.int32),
        pltpu.VMEM((8, LANES), jnp.float32),
        pltpu.VMEM((NS, 8, LANES), jnp.float32),
        pltpu.VMEM((LANES,), jnp.float32),
        pltpu.SemaphoreType.DMA,
        pltpu.SemaphoreType.DMA,
    ],
)
def _masked_loss(out_hbm, tgt_hbm, msk_hbm, part_hbm, loss_hbm, obuf, tbuf,
                 mbuf, pvec, pbuf, lbuf, sem0, sem1):
    c = lax.axis_index("c")
    s = lax.axis_index("s")
    b = c * B_PER_CORE + s // TILES_PER_B
    r_base = (s % TILES_PER_B) * ROWS_PER_TILE
    row = c * NS + s
    zero = jnp.zeros((LANES,), jnp.float32)
    one = jnp.ones((LANES,), jnp.float32)
    sems = (sem0, sem1)

    def start_fetch(ci):
        p = ci % 2
        sl = pl.ds(r_base + ci * CR, CR)
        return [
            pltpu.async_copy(out_hbm.at[b, sl, :], obuf.at[p], sems[p]),
            pltpu.async_copy(tgt_hbm.at[b, sl, :], tbuf.at[p], sems[p]),
            pltpu.async_copy(msk_hbm.at[b, sl, :], mbuf.at[p], sems[p]),
        ]

    inflight = start_fetch(0)
    acc = (zero,) * 8
    for ci in range(NCHUNK):
        p = ci % 2
        for cp in inflight:
            cp.wait()
        if ci + 1 < NCHUNK:
            inflight = start_fetch(ci + 1)

        def vec_body(j, carry, p=p):
            a1, a2, a3, a4, c1, c2, c3, c4 = carry
            i = j // 8
            c0 = (j % 8) * (LANES * UNROLL)
            for k in range(UNROLL):
                sl = pl.ds(c0 + k * LANES, LANES)
                o = obuf[p, i, sl]
                t = tbuf[p, i, sl]
                m = mbuf[p, i, sl]
                d = o - t
                d2 = d * d
                f1 = jnp.where(m == 1, one, zero)
                f2 = jnp.where(m == 2, one, zero)
                f3 = jnp.where(m == 3, one, zero)
                f4 = jnp.where(m == 4, one, zero)
                a1 = a1 + d2 * f1
                a2 = a2 + d2 * f2
                a3 = a3 + d2 * f3
                a4 = a4 + d2 * f4
                c1 = c1 + f1
                c2 = c2 + f2
                c3 = c3 + f3
                c4 = c4 + f4
            return (a1, a2, a3, a4, c1, c2, c3, c4)

        acc = lax.fori_loop(0, VPC, vec_body, acc)

    for k in range(8):
        pvec[k, :] = acc[k]
    pltpu.sync_copy(pvec, part_hbm.at[row])
    plsc.subcore_barrier()

    @pl.when(s == 0)
    def _():
        pltpu.sync_copy(part_hbm.at[pl.ds(c * NS, NS)], pbuf)
        lanes = lax.iota(jnp.int32, LANES)
        perms = [jnp.reshape(jnp.bitwise_xor(lanes, d), (LANES, 1))
                 for d in (1, 2, 4, 8)]
        dn = lax.GatherDimensionNumbers(
            offset_dims=(), collapsed_slice_dims=(0,), start_index_map=(0,))

        def lane_sum(v):
            for pm in perms:
                v = v + lax.gather(v, pm, dn, slice_sizes=(1,),
                                   mode=lax.GatherScatterMode.PROMISE_IN_BOUNDS)
            return v

        lossv = zero
        for bb in range(B_PER_CORE):
            t0 = TILES_PER_B * bb
            for i in range(4):
                v = (pbuf[t0 + 0, i, :] + pbuf[t0 + 1, i, :]
                     + pbuf[t0 + 2, i, :] + pbuf[t0 + 3, i, :])
                cc = (pbuf[t0 + 0, i + 4, :] + pbuf[t0 + 1, i + 4, :]
                      + pbuf[t0 + 2, i + 4, :] + pbuf[t0 + 3, i + 4, :])
                sv = lane_sum(v)
                cv = lane_sum(cc)
                contrib = jnp.where(cv > 0.0, sv / jnp.maximum(cv, 1.0), zero)
                lossv = lossv + contrib
        lbuf[...] = lossv * jnp.float32(1.0 / B)
        pltpu.sync_copy(lbuf, loss_hbm.at[c])


def kernel(output, target, mask):
    _, loss = _masked_loss(output, target, mask)
    return loss[0, 0] + loss[1, 0]


# speedup vs baseline: 2.4704x; 1.0642x over previous
import functools

import jax
import jax.numpy as jnp
from jax import lax
from jax.experimental import pallas as pl
from jax.experimental.pallas import tpu as pltpu
from jax.experimental.pallas import tpu_sc as plsc

B = 8
NC = 2
NS = 16
NW = NC * NS
B_PER_CORE = B // NC
TILES_PER_B = NS // B_PER_CORE
LANES = 16
ROWS = 512
COLS = 512
ROWS_PER_TILE = ROWS // TILES_PER_B
CR = 16
CHUNK = CR * COLS
NCHUNK = ROWS_PER_TILE // CR
UNROLL = 4
VPC = CHUNK // (LANES * UNROLL)

_mesh = plsc.VectorSubcoreMesh(core_axis_name="c", subcore_axis_name="s")


@functools.partial(
    pl.kernel,
    mesh=_mesh,
    out_type=(
        jax.ShapeDtypeStruct((NW, 8, LANES), jnp.float32),
        jax.ShapeDtypeStruct((NC, LANES), jnp.float32),
    ),
    compiler_params=pltpu.CompilerParams(
        needs_layout_passes=False, use_tc_tiling_on_sc=True),
    scratch_types=[
        pltpu.VMEM((2, CR, COLS), jnp.float32),
        pltpu.VMEM((2, CR, COLS), jnp.float32),
        pltpu.VMEM((2, CR, COLS), jnp.int32),
        pltpu.VMEM((8, LANES), jnp.float32),
        pltpu.VMEM((NS, 8, LANES), jnp.float32),
        pltpu.VMEM((LANES,), jnp.float32),
        pltpu.SemaphoreType.DMA,
        pltpu.SemaphoreType.DMA,
    ],
)
def _masked_loss(out_hbm, tgt_hbm, msk_hbm, part_hbm, loss_hbm, obuf, tbuf,
                 mbuf, pvec, pbuf, lbuf, sem0, sem1):
    c = lax.axis_index("c")
    s = lax.axis_index("s")
    b = c * B_PER_CORE + s // TILES_PER_B
    r_base = (s % TILES_PER_B) * ROWS_PER_TILE
    row = c * NS + s
    zero = jnp.zeros((LANES,), jnp.float32)
    one = jnp.ones((LANES,), jnp.float32)
    sems = (sem0, sem1)
    lanes0 = lax.iota(jnp.int32, LANES)
    ohs = [jnp.where(lanes0 == i, one, zero) for i in range(1, 5)]
    dn0 = lax.GatherDimensionNumbers(
        offset_dims=(), collapsed_slice_dims=(0,), start_index_map=(0,))

    def onehot(tbl, mm):
        return lax.gather(tbl, mm, dn0, slice_sizes=(1,),
                          mode=lax.GatherScatterMode.PROMISE_IN_BOUNDS)

    def start_fetch(ci):
        p = ci % 2
        sl = pl.ds(r_base + ci * CR, CR)
        return [
            pltpu.async_copy(out_hbm.at[b, sl, :], obuf.at[p], sems[p]),
            pltpu.async_copy(tgt_hbm.at[b, sl, :], tbuf.at[p], sems[p]),
            pltpu.async_copy(msk_hbm.at[b, sl, :], mbuf.at[p], sems[p]),
        ]

    inflight = start_fetch(0)
    acc = (zero,) * 8
    for ci in range(NCHUNK):
        p = ci % 2
        for cp in inflight:
            cp.wait()
        if ci + 1 < NCHUNK:
            inflight = start_fetch(ci + 1)

        def vec_body(j, carry, p=p):
            a1, a2, a3, a4, c1, c2, c3, c4 = carry
            i = j // 8
            c0 = (j % 8) * (LANES * UNROLL)
            for k in range(UNROLL):
                sl = pl.ds(c0 + k * LANES, LANES)
                o = obuf[p, i, sl]
                t = tbuf[p, i, sl]
                m = mbuf[p, i, sl]
                d = o - t
                d2 = d * d
                mm = jnp.reshape(m, (LANES, 1))
                f1 = onehot(ohs[0], mm)
                f2 = onehot(ohs[1], mm)
                f3 = onehot(ohs[2], mm)
                f4 = onehot(ohs[3], mm)
                a1 = a1 + d2 * f1
                a2 = a2 + d2 * f2
                a3 = a3 + d2 * f3
                a4 = a4 + d2 * f4
                c1 = c1 + f1
                c2 = c2 + f2
                c3 = c3 + f3
                c4 = c4 + f4
            return (a1, a2, a3, a4, c1, c2, c3, c4)

        acc = lax.fori_loop(0, VPC, vec_body, acc)

    for k in range(8):
        pvec[k, :] = acc[k]
    pltpu.sync_copy(pvec, part_hbm.at[row])
    plsc.subcore_barrier()

    @pl.when(s == 0)
    def _():
        pltpu.sync_copy(part_hbm.at[pl.ds(c * NS, NS)], pbuf)
        lanes = lax.iota(jnp.int32, LANES)
        perms = [jnp.reshape(jnp.bitwise_xor(lanes, d), (LANES, 1))
                 for d in (1, 2, 4, 8)]
        dn = lax.GatherDimensionNumbers(
            offset_dims=(), collapsed_slice_dims=(0,), start_index_map=(0,))

        def lane_sum(v):
            for pm in perms:
                v = v + lax.gather(v, pm, dn, slice_sizes=(1,),
                                   mode=lax.GatherScatterMode.PROMISE_IN_BOUNDS)
            return v

        lossv = zero
        for bb in range(B_PER_CORE):
            t0 = TILES_PER_B * bb
            for i in range(4):
                v = (pbuf[t0 + 0, i, :] + pbuf[t0 + 1, i, :]
                     + pbuf[t0 + 2, i, :] + pbuf[t0 + 3, i, :])
                cc = (pbuf[t0 + 0, i + 4, :] + pbuf[t0 + 1, i + 4, :]
                      + pbuf[t0 + 2, i + 4, :] + pbuf[t0 + 3, i + 4, :])
                sv = lane_sum(v)
                cv = lane_sum(cc)
                contrib = jnp.where(cv > 0.0, sv / jnp.maximum(cv, 1.0), zero)
                lossv = lossv + contrib
        lbuf[...] = lossv * jnp.float32(1.0 / B)
        pltpu.sync_copy(lbuf, loss_hbm.at[c])


def kernel(output, target, mask):
    _, loss = _masked_loss(output, target, mask)
    return loss[0, 0] + loss[1, 0]


# speedup vs baseline: 2.4781x; 1.0031x over previous
import functools

import jax
import jax.numpy as jnp
from jax import lax
from jax.experimental import pallas as pl
from jax.experimental.pallas import tpu as pltpu
from jax.experimental.pallas import tpu_sc as plsc

B = 8
NC = 2
NS = 16
NW = NC * NS
B_PER_CORE = B // NC
TILES_PER_B = NS // B_PER_CORE
LANES = 16
ROWS = 512
COLS = 512
ROWS_PER_TILE = ROWS // TILES_PER_B
CR = 16
CHUNK = CR * COLS
NCHUNK = ROWS_PER_TILE // CR
UNROLL = 4
VPC = CHUNK // (LANES * UNROLL)

_mesh = plsc.VectorSubcoreMesh(core_axis_name="c", subcore_axis_name="s")


@functools.partial(
    pl.kernel,
    mesh=_mesh,
    out_type=(
        jax.ShapeDtypeStruct((NW, 8, LANES), jnp.float32),
        jax.ShapeDtypeStruct((NC, LANES), jnp.float32),
    ),
    compiler_params=pltpu.CompilerParams(
        needs_layout_passes=False, use_tc_tiling_on_sc=True,
        disable_bounds_checks=True, disable_semaphore_checks=True,
        skip_device_barrier=True),
    scratch_types=[
        pltpu.VMEM((2, CR, COLS), jnp.float32),
        pltpu.VMEM((2, CR, COLS), jnp.float32),
        pltpu.VMEM((2, CR, COLS), jnp.int32),
        pltpu.VMEM((8, LANES), jnp.float32),
        pltpu.VMEM((NS, 8, LANES), jnp.float32),
        pltpu.VMEM((LANES,), jnp.float32),
        pltpu.SemaphoreType.DMA,
        pltpu.SemaphoreType.DMA,
    ],
)
def _masked_loss(out_hbm, tgt_hbm, msk_hbm, part_hbm, loss_hbm, obuf, tbuf,
                 mbuf, pvec, pbuf, lbuf, sem0, sem1):
    c = lax.axis_index("c")
    s = lax.axis_index("s")
    b = c * B_PER_CORE + s // TILES_PER_B
    r_base = (s % TILES_PER_B) * ROWS_PER_TILE
    row = c * NS + s
    zero = jnp.zeros((LANES,), jnp.float32)
    one = jnp.ones((LANES,), jnp.float32)
    sems = (sem0, sem1)
    lanes0 = lax.iota(jnp.int32, LANES)
    ohs = [jnp.where(lanes0 == i, one, zero) for i in range(1, 5)]
    dn0 = lax.GatherDimensionNumbers(
        offset_dims=(), collapsed_slice_dims=(0,), start_index_map=(0,))

    def onehot(tbl, mm):
        return lax.gather(tbl, mm, dn0, slice_sizes=(1,),
                          mode=lax.GatherScatterMode.PROMISE_IN_BOUNDS)

    def start_fetch(ci):
        p = ci % 2
        sl = pl.ds(r_base + ci * CR, CR)
        return [
            pltpu.async_copy(out_hbm.at[b, sl, :], obuf.at[p], sems[p]),
            pltpu.async_copy(tgt_hbm.at[b, sl, :], tbuf.at[p], sems[p]),
            pltpu.async_copy(msk_hbm.at[b, sl, :], mbuf.at[p], sems[p]),
        ]

    inflight = start_fetch(0)
    acc = (zero,) * 8
    for ci in range(NCHUNK):
        p = ci % 2
        for cp in inflight:
            cp.wait()
        if ci + 1 < NCHUNK:
            inflight = start_fetch(ci + 1)

        def vec_body(j, carry, p=p):
            a1, a2, a3, a4, c1, c2, c3, c4 = carry
            i = j // 8
            c0 = (j % 8) * (LANES * UNROLL)
            for k in range(UNROLL):
                sl = pl.ds(c0 + k * LANES, LANES)
                o = obuf[p, i, sl]
                t = tbuf[p, i, sl]
                m = mbuf[p, i, sl]
                d = o - t
                d2 = d * d
                mm = jnp.reshape(m, (LANES, 1))
                f1 = onehot(ohs[0], mm)
                f2 = onehot(ohs[1], mm)
                f3 = onehot(ohs[2], mm)
                f4 = onehot(ohs[3], mm)
                a1 = a1 + d2 * f1
                a2 = a2 + d2 * f2
                a3 = a3 + d2 * f3
                a4 = a4 + d2 * f4
                c1 = c1 + f1
                c2 = c2 + f2
                c3 = c3 + f3
                c4 = c4 + f4
            return (a1, a2, a3, a4, c1, c2, c3, c4)

        acc = lax.fori_loop(0, VPC, vec_body, acc)

    for k in range(8):
        pvec[k, :] = acc[k]
    pltpu.sync_copy(pvec, part_hbm.at[row])
    plsc.subcore_barrier()

    @pl.when(s == 0)
    def _():
        pltpu.sync_copy(part_hbm.at[pl.ds(c * NS, NS)], pbuf)
        lanes = lax.iota(jnp.int32, LANES)
        perms = [jnp.reshape(jnp.bitwise_xor(lanes, d), (LANES, 1))
                 for d in (1, 2, 4, 8)]
        dn = lax.GatherDimensionNumbers(
            offset_dims=(), collapsed_slice_dims=(0,), start_index_map=(0,))

        def lane_sum(v):
            for pm in perms:
                v = v + lax.gather(v, pm, dn, slice_sizes=(1,),
                                   mode=lax.GatherScatterMode.PROMISE_IN_BOUNDS)
            return v

        lossv = zero
        for bb in range(B_PER_CORE):
            t0 = TILES_PER_B * bb
            for i in range(4):
                v = (pbuf[t0 + 0, i, :] + pbuf[t0 + 1, i, :]
                     + pbuf[t0 + 2, i, :] + pbuf[t0 + 3, i, :])
                cc = (pbuf[t0 + 0, i + 4, :] + pbuf[t0 + 1, i + 4, :]
                      + pbuf[t0 + 2, i + 4, :] + pbuf[t0 + 3, i + 4, :])
                sv = lane_sum(v)
                cv = lane_sum(cc)
                contrib = jnp.where(cv > 0.0, sv / jnp.maximum(cv, 1.0), zero)
                lossv = lossv + contrib
        lbuf[...] = lossv * jnp.float32(1.0 / B)
        pltpu.sync_copy(lbuf, loss_hbm.at[c])


def kernel(output, target, mask):
    _, loss = _masked_loss(output, target, mask)
    return loss[0, 0] + loss[1, 0]
